# Initial kernel scaffold; baseline (speedup 1.0000x reference)
import jax
import jax.numpy as jnp
from jax.experimental import pallas as pl


def kernel(im_inds, obj_fmaps, obj_logits, rel_inds, vr, boxes_per_cls, W_op, b_op, W_rp, b_rp, W_emb, Wz_r, Uz_r, Wh_r, Uh_r, Wmsg, Wz_o, Uz_o, Wh_o, Uh_o, W_out, b_out):
    raise NotImplementedError("write your pallas kernel here")


# speedup vs baseline: 71.5432x; 71.5432x over previous
import functools

import jax
import jax.numpy as jnp
from jax.experimental import pallas as pl

N_OBJ = 1000
N_REL = 10000
C_OBJ = 151
C_REL = 51
H = 512
T = 3
REL_BLK = 1000
N_REL_BLKS = N_REL // REL_BLK
IOU_THR = 0.3


def _hobj_init_kernel(fmaps_ref, wop_ref, bop_ref, probs_ref, wemb_ref, out_ref):
    acc = jnp.dot(fmaps_ref[...], wop_ref[...], preferred_element_type=jnp.float32)
    emb = jnp.dot(probs_ref[...], wemb_ref[...], preferred_element_type=jnp.float32)
    out_ref[...] = acc + bop_ref[...] + emb


def _hobj_init(obj_fmaps, W_op, b_op, obj_probs, W_emb):
    return pl.pallas_call(
        _hobj_init_kernel,
        out_shape=jax.ShapeDtypeStruct((N_OBJ, H), jnp.float32),
    )(obj_fmaps, W_op, b_op.reshape(1, H), obj_probs, W_emb)


def _vr_proj_kernel(vr_ref, wrp_ref, brp_ref, out_ref):
    out_ref[...] = (
        jnp.dot(vr_ref[...], wrp_ref[...], preferred_element_type=jnp.float32)
        + brp_ref[...]
    )


def _vr_proj(vr, W_rp, b_rp):
    d = vr.shape[1]
    return pl.pallas_call(
        _vr_proj_kernel,
        grid=(N_REL_BLKS,),
        in_specs=[
            pl.BlockSpec((REL_BLK, d), lambda i: (i, 0)),
            pl.BlockSpec((d, H), lambda i: (0, 0)),
            pl.BlockSpec((1, H), lambda i: (0, 0)),
        ],
        out_specs=pl.BlockSpec((REL_BLK, H), lambda i: (i, 0)),
        out_shape=jax.ShapeDtypeStruct((N_REL, H), jnp.float32),
    )(vr, W_rp, b_rp.reshape(1, H))


def _rel_step_kernel(s_ref, o_ref, hrel_ref, hobj_ref, wz_ref, uz_ref,
                     wh_ref, uh_ref, wmsg_ref, hrel_out_ref, agg_ref):
    iota_n = jax.lax.broadcasted_iota(jnp.int32, (REL_BLK, N_OBJ), 1)
    oh_s = (s_ref[0] == iota_n).astype(jnp.float32)
    oh_o = (o_ref[0] == iota_n).astype(jnp.float32)
    hobj = hobj_ref[...]
    g_s = jnp.dot(oh_s, hobj, preferred_element_type=jnp.float32)
    g_o = jnp.dot(oh_o, hobj, preferred_element_type=jnp.float32)
    m = g_s * g_o
    h = hrel_ref[...]
    z = jax.nn.sigmoid(
        jnp.dot(m, wz_ref[...], preferred_element_type=jnp.float32)
        + jnp.dot(h, uz_ref[...], preferred_element_type=jnp.float32))
    h_new = (1.0 - z) * h + z * jnp.tanh(
        jnp.dot(m, wh_ref[...], preferred_element_type=jnp.float32)
        + jnp.dot(h, uh_ref[...], preferred_element_type=jnp.float32))
    hrel_out_ref[...] = h_new
    msg = jnp.dot(h_new, wmsg_ref[...], preferred_element_type=jnp.float32)
    part = jax.lax.dot_general(
        oh_s + oh_o, msg, (((0,), (0,)), ((), ())),
        preferred_element_type=jnp.float32)

    @pl.when(pl.program_id(0) == 0)
    def _init():
        agg_ref[...] = jnp.zeros_like(agg_ref)

    agg_ref[...] += part


def _rel_step(s3, o3, h_rel, h_obj, Wz_r, Uz_r, Wh_r, Uh_r, Wmsg):
    return pl.pallas_call(
        _rel_step_kernel,
        grid=(N_REL_BLKS,),
        in_specs=[
            pl.BlockSpec((1, REL_BLK, 1), lambda i: (i, 0, 0)),
            pl.BlockSpec((1, REL_BLK, 1), lambda i: (i, 0, 0)),
            pl.BlockSpec((REL_BLK, H), lambda i: (i, 0)),
            pl.BlockSpec((N_OBJ, H), lambda i: (0, 0)),
            pl.BlockSpec((H, H), lambda i: (0, 0)),
            pl.BlockSpec((H, H), lambda i: (0, 0)),
            pl.BlockSpec((H, H), lambda i: (0, 0)),
            pl.BlockSpec((H, H), lambda i: (0, 0)),
            pl.BlockSpec((H, H), lambda i: (0, 0)),
        ],
        out_specs=[
            pl.BlockSpec((REL_BLK, H), lambda i: (i, 0)),
            pl.BlockSpec((N_OBJ, H), lambda i: (0, 0)),
        ],
        out_shape=[
            jax.ShapeDtypeStruct((N_REL, H), jnp.float32),
            jax.ShapeDtypeStruct((N_OBJ, H), jnp.float32),
        ],
    )(s3, o3, h_rel, h_obj, Wz_r, Uz_r, Wh_r, Uh_r, Wmsg)


def _obj_step_kernel(agg_ref, hobj_ref, wz_ref, uz_ref, wh_ref, uh_ref, out_ref):
    agg = agg_ref[...]
    h = hobj_ref[...]
    z = jax.nn.sigmoid(
        jnp.dot(agg, wz_ref[...], preferred_element_type=jnp.float32)
        + jnp.dot(h, uz_ref[...], preferred_element_type=jnp.float32))
    out_ref[...] = (1.0 - z) * h + z * jnp.tanh(
        jnp.dot(agg, wh_ref[...], preferred_element_type=jnp.float32)
        + jnp.dot(h, uh_ref[...], preferred_element_type=jnp.float32))


def _obj_step(agg, h_obj, Wz_o, Uz_o, Wh_o, Uh_o):
    return pl.pallas_call(
        _obj_step_kernel,
        out_shape=jax.ShapeDtypeStruct((N_OBJ, H), jnp.float32),
    )(agg, h_obj, Wz_o, Uz_o, Wh_o, Uh_o)


def _rel_out_kernel(hrel_ref, wout_ref, bout_ref, out_ref):
    out_ref[...] = (
        jnp.dot(hrel_ref[...], wout_ref[...], preferred_element_type=jnp.float32)
        + bout_ref[...]
    )


def _rel_out(h_rel, W_out, b_out):
    return pl.pallas_call(
        _rel_out_kernel,
        grid=(N_REL_BLKS,),
        in_specs=[
            pl.BlockSpec((REL_BLK, H), lambda i: (i, 0)),
            pl.BlockSpec((H, C_REL), lambda i: (0, 0)),
            pl.BlockSpec((1, C_REL), lambda i: (0, 0)),
        ],
        out_specs=pl.BlockSpec((REL_BLK, C_REL), lambda i: (i, 0)),
        out_shape=jax.ShapeDtypeStruct((N_REL, C_REL), jnp.float32),
    )(h_rel, W_out, b_out.reshape(1, C_REL))


def _nms_kernel(boxes_ref, boxesT_ref, sc_row_ref, sc_col_ref,
                best_ref, besti_ref):
    n = N_OBJ
    b = boxes_ref[0]
    bt = boxesT_ref[0]
    x1r, y1r, x2r, y2r = b[:, 0:1], b[:, 1:2], b[:, 2:3], b[:, 3:4]
    x1c, y1c, x2c, y2c = bt[0:1, :], bt[1:2, :], bt[2:3, :], bt[3:4, :]
    area_r = (x2r - x1r) * (y2r - y1r)
    area_c = (x2c - x1c) * (y2c - y1c)
    xx1 = jnp.maximum(x1r, x1c)
    yy1 = jnp.maximum(y1r, y1c)
    xx2 = jnp.minimum(x2r, x2c)
    yy2 = jnp.minimum(y2r, y2c)
    inter = jnp.clip(xx2 - xx1, 0.0) * jnp.clip(yy2 - yy1, 0.0)
    iou = inter / (area_r + area_c - inter + 1e-9)
    overlap = iou > IOU_THR

    s_row = sc_row_ref[0]
    s_col = sc_col_ref[0]
    ir = jax.lax.broadcasted_iota(jnp.int32, (n, n), 0)
    ic = jax.lax.broadcasted_iota(jnp.int32, (n, n), 1)
    prec_rc = (s_col > s_row) | ((s_col == s_row) & (ir < ic))
    m_rc = (overlap & prec_rc).astype(jnp.float32)
    m_cr = (overlap & jnp.logical_not(prec_rc) & (ir != ic)).astype(jnp.float32)

    def body(state):
        k_row, _, it = state
        sup_col = jnp.max(m_cr * k_row, axis=1, keepdims=True)
        k_col = (sup_col < 0.5).astype(jnp.float32)
        sup_row = jnp.max(m_rc * k_col, axis=0, keepdims=True)
        k_row_new = (sup_row < 0.5).astype(jnp.float32)
        changed = jnp.sum(jnp.abs(k_row_new - k_row)) > 0.0
        return k_row_new, changed, it + 1

    def cond(state):
        _, changed, it = state
        return changed & (it < 2 * n)

    k_row = jnp.ones((1, n), jnp.float32)
    k_row, _, _ = jax.lax.while_loop(cond, body, (k_row, True, 0))
    sup_col = jnp.max(m_cr * k_row, axis=1, keepdims=True)
    k_col = (sup_col < 0.5).astype(jnp.float32)

    val = k_col * s_col
    cls_id = pl.program_id(0) + 1

    @pl.when(pl.program_id(0) == 0)
    def _init():
        best_ref[...] = jnp.zeros_like(best_ref)
        besti_ref[...] = jnp.ones_like(besti_ref)

    better = val > best_ref[...]
    best_ref[...] = jnp.where(better, val, best_ref[...])
    besti_ref[...] = jnp.where(better, cls_id, besti_ref[...])


def _nms_preds(boxes_cls, boxesT_cls, scores_row, scores_col):
    n_cls = C_OBJ - 1
    best, besti = pl.pallas_call(
        _nms_kernel,
        grid=(n_cls,),
        in_specs=[
            pl.BlockSpec((1, N_OBJ, 4), lambda i: (i, 0, 0)),
            pl.BlockSpec((1, 4, N_OBJ), lambda i: (i, 0, 0)),
            pl.BlockSpec((1, 1, N_OBJ), lambda i: (i, 0, 0)),
            pl.BlockSpec((1, N_OBJ, 1), lambda i: (i, 0, 0)),
        ],
        out_specs=[
            pl.BlockSpec((N_OBJ, 1), lambda i: (0, 0)),
            pl.BlockSpec((N_OBJ, 1), lambda i: (0, 0)),
        ],
        out_shape=[
            jax.ShapeDtypeStruct((N_OBJ, 1), jnp.float32),
            jax.ShapeDtypeStruct((N_OBJ, 1), jnp.int32),
        ],
    )(boxes_cls, boxesT_cls, scores_row, scores_col)
    return besti


@functools.partial(jax.jit, static_argnums=())
def kernel(im_inds, obj_fmaps, obj_logits, rel_inds, vr, boxes_per_cls,
           W_op, b_op, W_rp, b_rp, W_emb, Wz_r, Uz_r, Wh_r, Uh_r, Wmsg,
           Wz_o, Uz_o, Wh_o, Uh_o, W_out, b_out):
    obj_probs = jax.nn.softmax(obj_logits, axis=1)

    h_obj = _hobj_init(obj_fmaps, W_op, b_op, obj_probs, W_emb)
    h_rel = _vr_proj(vr, W_rp, b_rp)

    s3 = rel_inds[:, 1].reshape(N_REL_BLKS, REL_BLK, 1)
    o3 = rel_inds[:, 2].reshape(N_REL_BLKS, REL_BLK, 1)
    for _ in range(T):
        h_rel, agg = _rel_step(s3, o3, h_rel, h_obj,
                               Wz_r, Uz_r, Wh_r, Uh_r, Wmsg)
        h_obj = _obj_step(agg, h_obj, Wz_o, Uz_o, Wh_o, Uh_o)

    rel_logits = _rel_out(h_rel, W_out, b_out)

    boxes_cls = jnp.transpose(boxes_per_cls[:, 1:, :], (1, 0, 2))
    boxesT_cls = jnp.transpose(boxes_per_cls[:, 1:, :], (1, 2, 0))
    scores_row = obj_probs[:, 1:].T.reshape(C_OBJ - 1, 1, N_OBJ)
    scores_col = obj_probs[:, 1:].T.reshape(C_OBJ - 1, N_OBJ, 1)
    besti = _nms_preds(boxes_cls, boxesT_cls, scores_row, scores_col)
    obj_preds = besti.reshape(N_OBJ).astype(jnp.int32)

    return (obj_logits, obj_preds, rel_logits)


# speedup vs baseline: 74.1316x; 1.0362x over previous
import functools

import jax
import jax.numpy as jnp
from jax.experimental import pallas as pl
from jax.experimental.pallas import tpu as pltpu

N_OBJ = 1000
N_REL = 10000
C_OBJ = 151
C_REL = 51
H = 512
T = 3
REL_BLK = 1000
N_REL_BLKS = N_REL // REL_BLK
IOU_THR = 0.3


def _hobj_init_kernel(fmaps_ref, wop_ref, bop_ref, probs_ref, wemb_ref, out_ref):
    acc = jnp.dot(fmaps_ref[...], wop_ref[...], preferred_element_type=jnp.float32)
    emb = jnp.dot(probs_ref[...], wemb_ref[...], preferred_element_type=jnp.float32)
    out_ref[...] = acc + bop_ref[...] + emb


def _hobj_init(obj_fmaps, W_op, b_op, obj_probs, W_emb):
    return pl.pallas_call(
        _hobj_init_kernel,
        out_shape=jax.ShapeDtypeStruct((N_OBJ, H), jnp.float32),
    )(obj_fmaps, W_op, b_op.reshape(1, H), obj_probs, W_emb)


def _vr_proj_kernel(vr_ref, wrp_ref, brp_ref, out_ref):
    out_ref[...] = (
        jnp.dot(vr_ref[...], wrp_ref[...], preferred_element_type=jnp.float32)
        + brp_ref[...]
    )


def _vr_proj(vr, W_rp, b_rp):
    d = vr.shape[1]
    return pl.pallas_call(
        _vr_proj_kernel,
        grid=(N_REL_BLKS,),
        in_specs=[
            pl.BlockSpec((REL_BLK, d), lambda i: (i, 0)),
            pl.BlockSpec((d, H), lambda i: (0, 0)),
            pl.BlockSpec((1, H), lambda i: (0, 0)),
        ],
        out_specs=pl.BlockSpec((REL_BLK, H), lambda i: (i, 0)),
        out_shape=jax.ShapeDtypeStruct((N_REL, H), jnp.float32),
        compiler_params=pltpu.CompilerParams(
            dimension_semantics=("parallel",)),
    )(vr, W_rp, b_rp.reshape(1, H))


def _rel_step_kernel(s_ref, o_ref, hrel_ref, hobj_ref, wz_ref, uz_ref,
                     wh_ref, uh_ref, wmsg_ref, hrel_out_ref, agg_ref):
    iota_n = jax.lax.broadcasted_iota(jnp.int32, (REL_BLK, N_OBJ), 1)
    oh_s = (s_ref[0] == iota_n).astype(jnp.float32)
    oh_o = (o_ref[0] == iota_n).astype(jnp.float32)
    hobj = hobj_ref[...]
    g_s = jnp.dot(oh_s, hobj, preferred_element_type=jnp.float32)
    g_o = jnp.dot(oh_o, hobj, preferred_element_type=jnp.float32)
    m = g_s * g_o
    h = hrel_ref[...]
    z = jax.nn.sigmoid(
        jnp.dot(m, wz_ref[...], preferred_element_type=jnp.float32)
        + jnp.dot(h, uz_ref[...], preferred_element_type=jnp.float32))
    h_new = (1.0 - z) * h + z * jnp.tanh(
        jnp.dot(m, wh_ref[...], preferred_element_type=jnp.float32)
        + jnp.dot(h, uh_ref[...], preferred_element_type=jnp.float32))
    hrel_out_ref[...] = h_new
    msg = jnp.dot(h_new, wmsg_ref[...], preferred_element_type=jnp.float32)
    part = jax.lax.dot_general(
        oh_s + oh_o, msg, (((0,), (0,)), ((), ())),
        preferred_element_type=jnp.float32)

    @pl.when(pl.program_id(0) == 0)
    def _init():
        agg_ref[...] = jnp.zeros_like(agg_ref)

    agg_ref[...] += part


def _rel_step(s3, o3, h_rel, h_obj, Wz_r, Uz_r, Wh_r, Uh_r, Wmsg):
    return pl.pallas_call(
        _rel_step_kernel,
        grid=(N_REL_BLKS,),
        in_specs=[
            pl.BlockSpec((1, REL_BLK, 1), lambda i: (i, 0, 0)),
            pl.BlockSpec((1, REL_BLK, 1), lambda i: (i, 0, 0)),
            pl.BlockSpec((REL_BLK, H), lambda i: (i, 0)),
            pl.BlockSpec((N_OBJ, H), lambda i: (0, 0)),
            pl.BlockSpec((H, H), lambda i: (0, 0)),
            pl.BlockSpec((H, H), lambda i: (0, 0)),
            pl.BlockSpec((H, H), lambda i: (0, 0)),
            pl.BlockSpec((H, H), lambda i: (0, 0)),
            pl.BlockSpec((H, H), lambda i: (0, 0)),
        ],
        out_specs=[
            pl.BlockSpec((REL_BLK, H), lambda i: (i, 0)),
            pl.BlockSpec((N_OBJ, H), lambda i: (0, 0)),
        ],
        out_shape=[
            jax.ShapeDtypeStruct((N_REL, H), jnp.float32),
            jax.ShapeDtypeStruct((N_OBJ, H), jnp.float32),
        ],
    )(s3, o3, h_rel, h_obj, Wz_r, Uz_r, Wh_r, Uh_r, Wmsg)


def _obj_step_kernel(agg_ref, hobj_ref, wz_ref, uz_ref, wh_ref, uh_ref, out_ref):
    agg = agg_ref[...]
    h = hobj_ref[...]
    z = jax.nn.sigmoid(
        jnp.dot(agg, wz_ref[...], preferred_element_type=jnp.float32)
        + jnp.dot(h, uz_ref[...], preferred_element_type=jnp.float32))
    out_ref[...] = (1.0 - z) * h + z * jnp.tanh(
        jnp.dot(agg, wh_ref[...], preferred_element_type=jnp.float32)
        + jnp.dot(h, uh_ref[...], preferred_element_type=jnp.float32))


def _obj_step(agg, h_obj, Wz_o, Uz_o, Wh_o, Uh_o):
    return pl.pallas_call(
        _obj_step_kernel,
        out_shape=jax.ShapeDtypeStruct((N_OBJ, H), jnp.float32),
    )(agg, h_obj, Wz_o, Uz_o, Wh_o, Uh_o)


def _rel_out_kernel(hrel_ref, wout_ref, bout_ref, out_ref):
    out_ref[...] = (
        jnp.dot(hrel_ref[...], wout_ref[...], preferred_element_type=jnp.float32)
        + bout_ref[...]
    )


def _rel_out(h_rel, W_out, b_out):
    return pl.pallas_call(
        _rel_out_kernel,
        grid=(N_REL_BLKS,),
        in_specs=[
            pl.BlockSpec((REL_BLK, H), lambda i: (i, 0)),
            pl.BlockSpec((H, C_REL), lambda i: (0, 0)),
            pl.BlockSpec((1, C_REL), lambda i: (0, 0)),
        ],
        out_specs=pl.BlockSpec((REL_BLK, C_REL), lambda i: (i, 0)),
        out_shape=jax.ShapeDtypeStruct((N_REL, C_REL), jnp.float32),
        compiler_params=pltpu.CompilerParams(
            dimension_semantics=("parallel",)),
    )(h_rel, W_out, b_out.reshape(1, C_REL))


def _nms_kernel(boxes_ref, boxesT_ref, sc_row_ref, sc_col_ref, val_ref):
    n = N_OBJ
    b = boxes_ref[0]
    bt = boxesT_ref[0]
    x1r, y1r, x2r, y2r = b[:, 0:1], b[:, 1:2], b[:, 2:3], b[:, 3:4]
    x1c, y1c, x2c, y2c = bt[0:1, :], bt[1:2, :], bt[2:3, :], bt[3:4, :]
    area_r = (x2r - x1r) * (y2r - y1r)
    area_c = (x2c - x1c) * (y2c - y1c)
    xx1 = jnp.maximum(x1r, x1c)
    yy1 = jnp.maximum(y1r, y1c)
    xx2 = jnp.minimum(x2r, x2c)
    yy2 = jnp.minimum(y2r, y2c)
    inter = jnp.clip(xx2 - xx1, 0.0) * jnp.clip(yy2 - yy1, 0.0)
    iou = inter / (area_r + area_c - inter + 1e-9)
    overlap = iou > IOU_THR

    s_row = sc_row_ref[0]
    s_col = sc_col_ref[0]
    ir = jax.lax.broadcasted_iota(jnp.int32, (n, n), 0)
    ic = jax.lax.broadcasted_iota(jnp.int32, (n, n), 1)
    prec_rc = (s_col > s_row) | ((s_col == s_row) & (ir < ic))
    m_rc = (overlap & prec_rc).astype(jnp.bfloat16)

    def body(state):
        k_row, _, it = state
        sup = jax.lax.dot_general(
            k_row.astype(jnp.bfloat16), m_rc, (((1,), (0,)), ((), ())),
            preferred_element_type=jnp.float32)
        k_new = (sup < 0.5).astype(jnp.float32)
        changed = jnp.sum(jnp.abs(k_new - k_row)) > 0.0
        return k_new, changed, it + 1

    def cond(state):
        _, changed, it = state
        return changed & (it < n)

    k_row = jnp.ones((1, n), jnp.float32)
    k_row, _, _ = jax.lax.while_loop(cond, body, (k_row, True, 0))
    val_ref[0] = k_row * s_row


def _argmax_kernel(vals_ref, besti_ref):
    n_cls = C_OBJ - 1

    def body(c, state):
        best, besti = state
        v = vals_ref[pl.ds(c, 1), :]
        better = v > best
        best = jnp.where(better, v, best)
        besti = jnp.where(better, c + 1, besti)
        return best, besti

    best0 = jnp.zeros((1, N_OBJ), jnp.float32)
    besti0 = jnp.ones((1, N_OBJ), jnp.int32)
    _, besti = jax.lax.fori_loop(0, n_cls, body, (best0, besti0))
    besti_ref[...] = besti


def _nms_preds(boxes_cls, boxesT_cls, scores_row, scores_col):
    n_cls = C_OBJ - 1
    vals = pl.pallas_call(
        _nms_kernel,
        grid=(n_cls,),
        in_specs=[
            pl.BlockSpec((1, N_OBJ, 4), lambda i: (i, 0, 0)),
            pl.BlockSpec((1, 4, N_OBJ), lambda i: (i, 0, 0)),
            pl.BlockSpec((1, 1, N_OBJ), lambda i: (i, 0, 0)),
            pl.BlockSpec((1, N_OBJ, 1), lambda i: (i, 0, 0)),
        ],
        out_specs=pl.BlockSpec((1, 1, N_OBJ), lambda i: (i, 0, 0)),
        out_shape=jax.ShapeDtypeStruct((n_cls, 1, N_OBJ), jnp.float32),
        compiler_params=pltpu.CompilerParams(
            dimension_semantics=("parallel",)),
    )(boxes_cls, boxesT_cls, scores_row, scores_col)
    besti = pl.pallas_call(
        _argmax_kernel,
        out_shape=jax.ShapeDtypeStruct((1, N_OBJ), jnp.int32),
    )(vals.reshape(n_cls, N_OBJ))
    return besti


@functools.partial(jax.jit, static_argnums=())
def kernel(im_inds, obj_fmaps, obj_logits, rel_inds, vr, boxes_per_cls,
           W_op, b_op, W_rp, b_rp, W_emb, Wz_r, Uz_r, Wh_r, Uh_r, Wmsg,
           Wz_o, Uz_o, Wh_o, Uh_o, W_out, b_out):
    obj_probs = jax.nn.softmax(obj_logits, axis=1)

    h_obj = _hobj_init(obj_fmaps, W_op, b_op, obj_probs, W_emb)
    h_rel = _vr_proj(vr, W_rp, b_rp)

    s3 = rel_inds[:, 1].reshape(N_REL_BLKS, REL_BLK, 1)
    o3 = rel_inds[:, 2].reshape(N_REL_BLKS, REL_BLK, 1)
    for _ in range(T):
        h_rel, agg = _rel_step(s3, o3, h_rel, h_obj,
                               Wz_r, Uz_r, Wh_r, Uh_r, Wmsg)
        h_obj = _obj_step(agg, h_obj, Wz_o, Uz_o, Wh_o, Uh_o)

    rel_logits = _rel_out(h_rel, W_out, b_out)

    boxes_cls = jnp.transpose(boxes_per_cls[:, 1:, :], (1, 0, 2))
    boxesT_cls = jnp.transpose(boxes_per_cls[:, 1:, :], (1, 2, 0))
    scores_row = obj_probs[:, 1:].T.reshape(C_OBJ - 1, 1, N_OBJ)
    scores_col = obj_probs[:, 1:].T.reshape(C_OBJ - 1, N_OBJ, 1)
    besti = _nms_preds(boxes_cls, boxesT_cls, scores_row, scores_col)
    obj_preds = besti.reshape(N_OBJ)

    return (obj_logits, obj_preds, rel_logits)
